# baseline (speedup 1.0000x reference)
import jax
import jax.numpy as jnp
from jax import lax
from jax.experimental import pallas as pl
from jax.experimental.pallas import tpu as pltpu

_VIEW_ROWS = 125000
_NCHUNK = 25
_BLOCK = _VIEW_ROWS // _NCHUNK
_NBUF = 4
_LOOKAHEAD = 2


def _copy_kernel(x, o, vmem, in_sems, out_sems):

    def in_dma(i, buf):
        return pltpu.make_async_copy(
            x.at[pl.ds(i * _BLOCK, _BLOCK)], vmem.at[buf], in_sems.at[buf]
        )

    def out_dma(i, buf):
        return pltpu.make_async_copy(
            vmem.at[buf], o.at[pl.ds(i * _BLOCK, _BLOCK)], out_sems.at[buf]
        )

    for j in range(_LOOKAHEAD):
        in_dma(j, j).start()

    def body(i, _):
        j = i + _LOOKAHEAD
        @pl.when(j < _NCHUNK)
        def _():
            jbuf = lax.rem(j, _NBUF)
            @pl.when(j >= _NBUF)
            def _():
                out_dma(j - _NBUF, jbuf).wait()
            in_dma(j, jbuf).start()
        buf = lax.rem(i, _NBUF)
        in_dma(i, buf).wait()
        out_dma(i, buf).start()
        return 0

    lax.fori_loop(0, _NCHUNK, body, 0)
    for j in range(_NCHUNK - _NBUF, _NCHUNK):
        out_dma(j, j % _NBUF).wait()


def kernel(embeddings):
    n, d = embeddings.shape
    x = embeddings.reshape(_VIEW_ROWS, 128)
    out = pl.pallas_call(
        _copy_kernel,
        in_specs=[pl.BlockSpec(memory_space=pltpu.MemorySpace.HBM)],
        out_specs=pl.BlockSpec(memory_space=pltpu.MemorySpace.HBM),
        out_shape=jax.ShapeDtypeStruct((_VIEW_ROWS, 128), embeddings.dtype),
        scratch_shapes=[
            pltpu.VMEM((_NBUF, _BLOCK, 128), jnp.float32),
            pltpu.SemaphoreType.DMA((_NBUF,)),
            pltpu.SemaphoreType.DMA((_NBUF,)),
        ],
    )(x)
    return out.reshape(n, d)
